# speedup vs baseline: 73.0542x; 73.0542x over previous
import functools

import jax
import jax.numpy as jnp
from jax import lax
from jax.experimental import pallas as pl

POOL = 1000000
D = 64
QN = 64
K = 8
BLK = 8192
NB = 123
PPAD = NB * BLK
NSEG = NB * 128
SEGW = BLK // 128
NCAND = K * SEGW
NEG = -1e30


def _k1_body(q_ref, kb_ref, out_ref):
    b = pl.program_id(0)
    q = q_ref[...]
    kb = kb_ref[...]
    scores = lax.dot_general(q, kb, (((1,), (1,)), ((), ())),
                             preferred_element_type=jnp.float32)
    s3 = scores.reshape(QN, SEGW, 128)
    out_ref[...] = jnp.max(s3, axis=1)

    @pl.when(b == NB - 1)
    def _():
        pos = lax.broadcasted_iota(jnp.int32, (QN, SEGW, 128), 1) * 128 + \
              lax.broadcasted_iota(jnp.int32, (QN, SEGW, 128), 2)
        valid = (b * BLK + pos) < POOL
        out_ref[...] = jnp.max(jnp.where(valid, s3, NEG), axis=1)


def _k2_body(seg_ref, cand_ref):
    x = seg_ref[...]
    iota = lax.broadcasted_iota(jnp.int32, (QN, NSEG), 1)
    ids = []
    for _ in range(K):
        m = jnp.max(x, axis=1, keepdims=True)
        sel = x == m
        idx = jnp.min(jnp.where(sel, iota, jnp.int32(2**30)), axis=1,
                      keepdims=True)
        ids.append(idx)
        x = jnp.where(iota == idx, NEG, x)
    seg_ids = jnp.concatenate(ids, axis=1)
    blk = seg_ids // 128
    lane = seg_ids % 128
    base = blk * BLK + lane
    j = lax.broadcasted_iota(jnp.int32, (QN, K, SEGW), 2)
    cand_ref[...] = (base[:, :, None] + j * 128).reshape(QN, NCAND)


def _k4_body(q_ref, kc_ref, ci_ref, idx_ref, w_ref):
    q = q_ref[...]
    kc = kc_ref[...]
    ci = ci_ref[...]
    scores = jnp.sum(kc * q[:, None, :], axis=-1)
    scores = jnp.where(ci < POOL, scores, NEG)
    vals = []
    idxs = []
    for _ in range(K):
        m = jnp.max(scores, axis=1, keepdims=True)
        sel = scores == m
        pidx = jnp.min(jnp.where(sel, ci, jnp.int32(2**30)), axis=1,
                       keepdims=True)
        vals.append(m)
        idxs.append(pidx)
        scores = jnp.where(ci == pidx, NEG, scores)
    v = jnp.concatenate(vals, axis=1)
    idx_ref[...] = jnp.concatenate(idxs, axis=1)
    e = jnp.exp(v - v[:, 0:1])
    w_ref[...] = e / jnp.sum(e, axis=1, keepdims=True)


def _k6_body(rows_ref, w_ref, W_ref, out_ref):
    rows = rows_ref[...]
    w = w_ref[...]
    agg = jnp.sum(rows * w[:, :, None], axis=1)
    out_ref[...] = lax.dot_general(agg, W_ref[...], (((1,), (1,)), ((), ())),
                                   preferred_element_type=jnp.float32)


@jax.jit
def kernel(query, pool, keys, W):
    B, S, _ = query.shape
    q = query.reshape(QN, D)
    keys_pad = jnp.pad(keys, ((0, PPAD - POOL), (0, 0)))

    seg_max = pl.pallas_call(
        _k1_body,
        grid=(NB,),
        in_specs=[
            pl.BlockSpec((QN, D), lambda b: (0, 0)),
            pl.BlockSpec((BLK, D), lambda b: (b, 0)),
        ],
        out_specs=pl.BlockSpec((QN, 128), lambda b: (0, b)),
        out_shape=jax.ShapeDtypeStruct((QN, NSEG), jnp.float32),
    )(q, keys_pad)

    cand_idx = pl.pallas_call(
        _k2_body,
        out_shape=jax.ShapeDtypeStruct((QN, NCAND), jnp.int32),
    )(seg_max)

    kc = jnp.take(keys_pad, cand_idx.reshape(-1), axis=0).reshape(QN, NCAND, D)

    final_idx, weights = pl.pallas_call(
        _k4_body,
        out_shape=(jax.ShapeDtypeStruct((QN, K), jnp.int32),
                   jax.ShapeDtypeStruct((QN, K), jnp.float32)),
    )(q, kc, cand_idx)

    rows = jnp.take(pool, final_idx.reshape(-1), axis=0).reshape(QN, K, D)

    out = pl.pallas_call(
        _k6_body,
        out_shape=jax.ShapeDtypeStruct((QN, D), jnp.float32),
    )(rows, weights, W)
    return out.reshape(B, S, D)


# speedup vs baseline: 91.7791x; 1.2563x over previous
import functools

import jax
import jax.numpy as jnp
from jax import lax
from jax.experimental import pallas as pl

POOL = 1000000
D = 64
QN = 64
K = 8
BLK = 8192
NB = 123
PPAD = NB * BLK
NSEG = NB * 128
SEGW = BLK // 128
NCAND = K * SEGW
NEG = -1e30


def _k1_body(q_ref, kb_ref, out_ref):
    b = pl.program_id(0)
    q = q_ref[...]
    kb = kb_ref[...]
    scores = lax.dot_general(q, kb, (((1,), (1,)), ((), ())),
                             preferred_element_type=jnp.float32)
    s3 = scores.reshape(QN, SEGW, 128)
    out_ref[...] = jnp.max(s3, axis=1)

    @pl.when(b == NB - 1)
    def _():
        pos = lax.broadcasted_iota(jnp.int32, (QN, SEGW, 128), 1) * 128 + \
              lax.broadcasted_iota(jnp.int32, (QN, SEGW, 128), 2)
        valid = (b * BLK + pos) < POOL
        out_ref[...] = jnp.max(jnp.where(valid, s3, NEG), axis=1)


def _k2_body(seg_ref, cand_ref):
    x = seg_ref[...]
    iota = lax.broadcasted_iota(jnp.int32, (QN, NSEG), 1)
    ids = []
    for _ in range(K):
        m = jnp.max(x, axis=1, keepdims=True)
        sel = x == m
        idx = jnp.min(jnp.where(sel, iota, jnp.int32(2**30)), axis=1,
                      keepdims=True)
        ids.append(idx)
        x = jnp.where(iota == idx, NEG, x)
    seg_ids = jnp.concatenate(ids, axis=1)
    blk = seg_ids // 128
    lane = seg_ids % 128
    base = blk * BLK + lane
    j = lax.broadcasted_iota(jnp.int32, (QN, K, SEGW), 2)
    cand_ref[...] = (base[:, :, None] + j * 128).reshape(QN, NCAND)


def _k4_body(q_ref, kc_ref, ci_ref, idx_ref, w_ref):
    q = q_ref[...]
    kc = kc_ref[...]
    ci = ci_ref[...]
    scores = jnp.sum(kc * q[:, None, :], axis=-1)
    scores = jnp.where(ci < POOL, scores, NEG)
    vals = []
    idxs = []
    for _ in range(K):
        m = jnp.max(scores, axis=1, keepdims=True)
        sel = scores == m
        pidx = jnp.min(jnp.where(sel, ci, jnp.int32(2**30)), axis=1,
                       keepdims=True)
        vals.append(m)
        idxs.append(pidx)
        scores = jnp.where(ci == pidx, NEG, scores)
    v = jnp.concatenate(vals, axis=1)
    idx_ref[...] = jnp.concatenate(idxs, axis=1)
    e = jnp.exp(v - v[:, 0:1])
    w_ref[...] = e / jnp.sum(e, axis=1, keepdims=True)


def _k6_body(rows_ref, w_ref, W_ref, out_ref):
    rows = rows_ref[...]
    w = w_ref[...]
    agg = jnp.sum(rows * w[:, :, None], axis=1)
    out_ref[...] = lax.dot_general(agg, W_ref[...], (((1,), (1,)), ((), ())),
                                   preferred_element_type=jnp.float32)


@jax.jit
def kernel(query, pool, keys, W):
    B, S, _ = query.shape
    q = query.reshape(QN, D)

    seg_max = pl.pallas_call(
        _k1_body,
        grid=(NB,),
        in_specs=[
            pl.BlockSpec((QN, D), lambda b: (0, 0)),
            pl.BlockSpec((BLK, D), lambda b: (b, 0)),
        ],
        out_specs=pl.BlockSpec((QN, 128), lambda b: (0, b)),
        out_shape=jax.ShapeDtypeStruct((QN, NSEG), jnp.float32),
    )(q, keys)

    cand_idx = pl.pallas_call(
        _k2_body,
        out_shape=jax.ShapeDtypeStruct((QN, NCAND), jnp.int32),
    )(seg_max)

    kc = jnp.take(keys, cand_idx.reshape(-1), axis=0,
                  mode="clip").reshape(QN, NCAND, D)

    final_idx, weights = pl.pallas_call(
        _k4_body,
        out_shape=(jax.ShapeDtypeStruct((QN, K), jnp.int32),
                   jax.ShapeDtypeStruct((QN, K), jnp.float32)),
    )(q, kc, cand_idx)

    rows = jnp.take(pool, final_idx.reshape(-1), axis=0).reshape(QN, K, D)

    out = pl.pallas_call(
        _k6_body,
        out_shape=jax.ShapeDtypeStruct((QN, D), jnp.float32),
    )(rows, weights, W)
    return out.reshape(B, S, D)
